# baseline (speedup 1.0000x reference)
import functools

import jax
import jax.numpy as jnp
from jax.experimental import pallas as pl
from jax.experimental.pallas import tpu as pltpu

M_TILE = 512
N_TILE = 2048


def _router_kernel(x_ref, w_ref, b_ref, emb_ref, out_ref, h_ref):
    n = pl.program_id(1)

    @pl.when(n == 0)
    def _():
        h_ref[...] = (
            jnp.dot(x_ref[...], w_ref[...], preferred_element_type=jnp.float32)
            + b_ref[...]
        )

    emb = emb_ref[...]
    inv = jax.lax.rsqrt(jnp.maximum(jnp.sum(emb * emb, axis=1, keepdims=True), 1e-24))
    emb_n = emb * inv
    out_ref[...] = jax.lax.dot_general(
        h_ref[...], emb_n,
        dimension_numbers=(((1,), (1,)), ((), ())),
        preferred_element_type=jnp.float32,
    )


@functools.partial(jax.jit, static_argnums=())
def kernel(x, W, b, neuron_emb):
    Bb, S, D = x.shape
    N, d_space = neuron_emb.shape
    M = Bb * S
    x2 = x.reshape(M, D)
    b2 = b.reshape(1, d_space)

    grid = (M // M_TILE, N // N_TILE)
    out = pl.pallas_call(
        _router_kernel,
        grid=grid,
        in_specs=[
            pl.BlockSpec((M_TILE, D), lambda m, n: (m, 0)),
            pl.BlockSpec((D, d_space), lambda m, n: (0, 0)),
            pl.BlockSpec((1, d_space), lambda m, n: (0, 0)),
            pl.BlockSpec((N_TILE, d_space), lambda m, n: (n, 0)),
        ],
        out_specs=pl.BlockSpec((M_TILE, N_TILE), lambda m, n: (m, n)),
        out_shape=jax.ShapeDtypeStruct((M, N), jnp.float32),
        scratch_shapes=[pltpu.VMEM((M_TILE, d_space), jnp.float32)],
        compiler_params=pltpu.CompilerParams(
            dimension_semantics=("arbitrary", "arbitrary"),
        ),
    )(x2, W, b2, neuron_emb)
    return out.reshape(Bb, S, N)


# speedup vs baseline: 1.5033x; 1.5033x over previous
import functools

import jax
import jax.numpy as jnp
from jax.experimental import pallas as pl
from jax.experimental.pallas import tpu as pltpu

M_TILE = 512


def _router_kernel(x_ref, w_ref, b_ref, emb_ref, out_ref, h_ref, embn_ref):
    m = pl.program_id(0)

    @pl.when(m == 0)
    def _():
        emb = emb_ref[...]
        inv = jax.lax.rsqrt(
            jnp.maximum(jnp.sum(emb * emb, axis=1, keepdims=True), 1e-24)
        )
        embn_ref[...] = emb * inv

    h_ref[...] = (
        jnp.dot(x_ref[...], w_ref[...], preferred_element_type=jnp.float32)
        + b_ref[...]
    )
    out_ref[...] = jax.lax.dot_general(
        h_ref[...], embn_ref[...],
        dimension_numbers=(((1,), (1,)), ((), ())),
        preferred_element_type=jnp.float32,
    )


@functools.partial(jax.jit, static_argnums=())
def kernel(x, W, b, neuron_emb):
    Bb, S, D = x.shape
    N, d_space = neuron_emb.shape
    M = Bb * S
    x2 = x.reshape(M, D)
    b2 = b.reshape(1, d_space)

    grid = (M // M_TILE,)
    out = pl.pallas_call(
        _router_kernel,
        grid=grid,
        in_specs=[
            pl.BlockSpec((M_TILE, D), lambda m: (m, 0)),
            pl.BlockSpec((D, d_space), lambda m: (0, 0)),
            pl.BlockSpec((1, d_space), lambda m: (0, 0)),
            pl.BlockSpec((N, d_space), lambda m: (0, 0)),
        ],
        out_specs=pl.BlockSpec((M_TILE, N), lambda m: (m, 0)),
        out_shape=jax.ShapeDtypeStruct((M, N), jnp.float32),
        scratch_shapes=[
            pltpu.VMEM((M_TILE, d_space), jnp.float32),
            pltpu.VMEM((N, d_space), jnp.float32),
        ],
        compiler_params=pltpu.CompilerParams(
            dimension_semantics=("arbitrary",),
        ),
    )(x2, W, b2, neuron_emb)
    return out.reshape(Bb, S, N)
